# baseline (speedup 1.0000x reference)
import functools

import jax
import jax.numpy as jnp
from jax import lax
from jax.experimental import pallas as pl
from jax.experimental.pallas import tpu as pltpu
from jax.experimental.pallas import tpu_sc as plsc

_NB = 256
_COMMIT = 0.25
_WIN = 2736
_PADW = 2816

_SC_CORES = 2
_SC_SUBCORES = 16
_SC_WORKERS = _SC_CORES * _SC_SUBCORES
_SC_CHUNK = 128


def _tc_vq_body(nsteps, n_total, d_dim, k_codes,
                xsq_ref, x_ref, wt_ref, wsq_ref, codes_ref, iotaf_ref,
                idx_ref, loss_ref, perp_ref, acc_ref, cnt_ref):
    i = pl.program_id(0)

    @pl.when(i == 0)
    def _init():
        acc_ref[0, 0] = 0.0
        cnt_ref[...] = jnp.zeros_like(cnt_ref)

    xt = x_ref[0]
    m2 = lax.dot_general(xt, wt_ref[...], (((0,), (0,)), ((), ())),
                         preferred_element_type=jnp.float32,
                         precision=lax.Precision.DEFAULT)
    scores = (xsq_ref[...] + wsq_ref[...]) - m2
    iota_f = iotaf_ref[...]

    def win_min(w):
        return jnp.min(scores[:, w * _PADW:(w + 1) * _PADW], axis=1,
                       keepdims=True)

    def win_idx(w, v):
        sw = scores[:, w * _PADW:(w + 1) * _PADW]
        loc_f = jnp.min(jnp.where(sw == v, iota_f, jnp.float32(_PADW)),
                        axis=1, keepdims=True)
        return loc_f.astype(jnp.int32) + (w * _WIN)

    v0 = win_min(0)
    v1 = win_min(1)
    v2 = win_min(2)

    def bf(v):
        return v.astype(jnp.bfloat16).astype(jnp.float32)

    acc_s = bf(v0)
    take1 = v1 < acc_s
    acc_s = jnp.where(take1, bf(v1), acc_s)
    acc_t = jnp.where(take1, v1, v0)
    take2 = v2 < acc_s
    acc_t = jnp.where(take2, v2, acc_t)

    w0 = win_idx(0, v0)
    w2 = win_idx(2, v2)
    need1 = jnp.any(take1 & jnp.logical_not(take2))
    w1 = lax.cond(need1, lambda: win_idx(1, v1), lambda: w0)
    idx = jnp.where(take1, w1, w0)
    idx = jnp.where(take2, w2, idx)

    idx_ref[...] = idx

    acc_ref[0, 0] += jnp.sum(acc_t)
    onehot = (codes_ref[...] == idx).astype(jnp.bfloat16)
    ones_row = jnp.ones((1, onehot.shape[0]), jnp.bfloat16)
    cnt_ref[...] += lax.dot_general(
        ones_row, onehot, (((1,), (0,)), ((), ())),
        preferred_element_type=jnp.float32)

    @pl.when(i == nsteps - 1)
    def _finalize():
        mse = acc_ref[0, 0] * (1.0 / (n_total * d_dim))
        loss_ref[0, 0] = mse + _COMMIT * mse
        p = cnt_ref[...] * (1.0 / n_total)
        ent = -jnp.sum(p * jnp.log(p + 1e-10))
        perp_ref[0, 0] = jnp.exp(ent)


def _tc_vq(x_nc, xsq, wt_pad, wsq_pad, codes, iotaf, k_codes):
    nb_img, d, hw = x_nc.shape
    n = nb_img * hw
    kp = wt_pad.shape[1]
    nsteps = n // _NB
    blocks_per_img = hw // _NB
    idx2, loss, perp = pl.pallas_call(
        functools.partial(_tc_vq_body, nsteps, n, d, k_codes),
        grid=(nsteps,),
        in_specs=[
            pl.BlockSpec((_NB, 1), lambda i: (i, 0)),
            pl.BlockSpec((1, d, _NB),
                         lambda i: (i // blocks_per_img, 0,
                                    i % blocks_per_img)),
            pl.BlockSpec((d, kp), lambda i: (0, 0)),
            pl.BlockSpec((1, kp), lambda i: (0, 0)),
            pl.BlockSpec((1, kp), lambda i: (0, 0)),
            pl.BlockSpec((1, _PADW), lambda i: (0, 0)),
        ],
        out_specs=[
            pl.BlockSpec((_NB, 1), lambda i: (i, 0)),
            pl.BlockSpec(memory_space=pltpu.SMEM),
            pl.BlockSpec(memory_space=pltpu.SMEM),
        ],
        out_shape=[
            jax.ShapeDtypeStruct((n, 1), jnp.int32),
            jax.ShapeDtypeStruct((1, 1), jnp.float32),
            jax.ShapeDtypeStruct((1, 1), jnp.float32),
        ],
        scratch_shapes=[
            pltpu.SMEM((1, 1), jnp.float32),
            pltpu.VMEM((1, kp), jnp.float32),
        ],
    )(xsq, x_nc, wt_pad, wsq_pad, codes, iotaf)
    return idx2.reshape(n), loss[0, 0], perp[0, 0]


def _sc_gather(table, idx):
    b = idx.shape[0]
    d = table.shape[1]
    b_per_w = b // _SC_WORKERS
    n_chunks = b_per_w // _SC_CHUNK
    mesh = plsc.VectorSubcoreMesh(core_axis_name="c", subcore_axis_name="s")

    @functools.partial(
        pl.kernel,
        out_type=jax.ShapeDtypeStruct((b, d), jnp.float32),
        mesh=mesh,
        scratch_types=[
            pltpu.VMEM((_SC_CHUNK,), jnp.int32),
            pltpu.VMEM((_SC_CHUNK, d), jnp.float32),
            pltpu.SemaphoreType.DMA,
        ],
    )
    def gather_kernel(idx_hbm, table_hbm, out_hbm, idx_v, rows_v, sem):
        wid = lax.axis_index("s") * _SC_CORES + lax.axis_index("c")
        base = wid * b_per_w
        for c in range(n_chunks):
            off = base + c * _SC_CHUNK
            pltpu.sync_copy(idx_hbm.at[pl.ds(off, _SC_CHUNK)], idx_v)
            pltpu.async_copy(table_hbm.at[idx_v], rows_v, sem).wait()
            pltpu.sync_copy(rows_v, out_hbm.at[pl.ds(off, _SC_CHUNK)])

    return gather_kernel(idx, table)


def kernel(inputs, W):
    k, d = W.shape
    x = jnp.transpose(inputs, (0, 2, 3, 1))
    x_flat = x.reshape(-1, d)
    x_nc = inputs.reshape(inputs.shape[0], d, -1)
    xsq = jnp.sum(x_flat ** 2, axis=1, keepdims=True)
    wsq = jnp.sum(W ** 2, axis=1)
    wt = 2.0 * W.T

    pads = (_PADW - _WIN, _PADW - _WIN, _PADW - (k - 2 * _WIN))
    wt_win, wsq_win, code_win = [], [], []
    for w in range(3):
        lo = w * _WIN
        hi = min((w + 1) * _WIN, k)
        wt_win.append(jnp.pad(wt[:, lo:hi], ((0, 0), (0, pads[w]))))
        wsq_win.append(jnp.pad(wsq[lo:hi], (0, pads[w]),
                               constant_values=jnp.inf))
        code_win.append(jnp.pad(jnp.arange(lo, hi, dtype=jnp.int32),
                                (0, pads[w]), constant_values=-1))
    wt_pad = jnp.concatenate(wt_win, axis=1)
    wsq_pad = jnp.concatenate(wsq_win).reshape(1, -1)
    codes = jnp.concatenate(code_win).reshape(1, -1)

    iotaf = jnp.arange(_PADW, dtype=jnp.float32).reshape(1, -1)
    idx, loss, perp = _tc_vq(x_nc, xsq, wt_pad, wsq_pad, codes, iotaf, k)
    quantized = _sc_gather(W, idx)
    q_nchw = jnp.transpose(quantized.reshape(x.shape), (0, 3, 1, 2))
    return loss, q_nchw, perp, idx


# speedup vs baseline: 1.0625x; 1.0625x over previous
import functools

import jax
import jax.numpy as jnp
from jax import lax
from jax.experimental import pallas as pl
from jax.experimental.pallas import tpu as pltpu
from jax.experimental.pallas import tpu_sc as plsc

_NB = 256
_COMMIT = 0.25
_WIN = 2736
_PADW = 2816

_SC_CORES = 2
_SC_SUBCORES = 16
_SC_WORKERS = _SC_CORES * _SC_SUBCORES
_SC_CHUNK = 128


def _tc_vq_body(nsteps, n_total, d_dim, k_codes,
                xsq_ref, x_ref, wt_ref, wsq_ref, codes_ref, iotaf_ref,
                idx_ref, loss_ref, perp_ref, acc_ref, cnt_ref):
    i = pl.program_id(0)

    @pl.when(i == 0)
    def _init():
        acc_ref[0, 0] = 0.0
        cnt_ref[...] = jnp.zeros_like(cnt_ref)

    x = x_ref[...]
    m2 = lax.dot_general(x, wt_ref[...], (((1,), (0,)), ((), ())),
                         preferred_element_type=jnp.float32,
                         precision=lax.Precision.DEFAULT)
    scores = (xsq_ref[...] + wsq_ref[...]) - m2
    iota_f = iotaf_ref[...]

    def win_min(w):
        return jnp.min(scores[:, w * _PADW:(w + 1) * _PADW], axis=1,
                       keepdims=True)

    def win_idx(w, v):
        sw = scores[:, w * _PADW:(w + 1) * _PADW]
        loc_f = jnp.min(jnp.where(sw == v, iota_f, jnp.float32(_PADW)),
                        axis=1, keepdims=True)
        return loc_f.astype(jnp.int32) + (w * _WIN)

    v0 = win_min(0)
    v1 = win_min(1)
    v2 = win_min(2)

    def bf(v):
        return v.astype(jnp.bfloat16).astype(jnp.float32)

    acc_s = bf(v0)
    take1 = v1 < acc_s
    acc_s = jnp.where(take1, bf(v1), acc_s)
    acc_t = jnp.where(take1, v1, v0)
    take2 = v2 < acc_s
    acc_t = jnp.where(take2, v2, acc_t)

    w0 = win_idx(0, v0)
    w2 = win_idx(2, v2)
    need1 = jnp.any(take1 & jnp.logical_not(take2))
    w1 = lax.cond(need1, lambda: win_idx(1, v1), lambda: w0)
    idx = jnp.where(take1, w1, w0)
    idx = jnp.where(take2, w2, idx)

    idx_ref[...] = idx

    acc_ref[0, 0] += jnp.sum(acc_t)
    onehot = (codes_ref[...] == idx).astype(jnp.bfloat16)
    ones_row = jnp.ones((1, onehot.shape[0]), jnp.bfloat16)
    cnt_ref[...] += lax.dot_general(
        ones_row, onehot, (((1,), (0,)), ((), ())),
        preferred_element_type=jnp.float32)

    @pl.when(i == nsteps - 1)
    def _finalize():
        mse = acc_ref[0, 0] * (1.0 / (n_total * d_dim))
        loss_ref[0, 0] = mse + _COMMIT * mse
        p = cnt_ref[...] * (1.0 / n_total)
        ent = -jnp.sum(p * jnp.log(p + 1e-10))
        perp_ref[0, 0] = jnp.exp(ent)


def _tc_vq(x, xsq, wt_pad, wsq_pad, codes, iotaf, k_codes):
    n, d = x.shape
    kp = wt_pad.shape[1]
    nsteps = n // _NB
    idx2, loss, perp = pl.pallas_call(
        functools.partial(_tc_vq_body, nsteps, n, d, k_codes),
        grid=(nsteps,),
        in_specs=[
            pl.BlockSpec((_NB, 1), lambda i: (i, 0)),
            pl.BlockSpec((_NB, d), lambda i: (i, 0)),
            pl.BlockSpec((d, kp), lambda i: (0, 0)),
            pl.BlockSpec((1, kp), lambda i: (0, 0)),
            pl.BlockSpec((1, kp), lambda i: (0, 0)),
            pl.BlockSpec((1, _PADW), lambda i: (0, 0)),
        ],
        out_specs=[
            pl.BlockSpec((_NB, 1), lambda i: (i, 0)),
            pl.BlockSpec(memory_space=pltpu.SMEM),
            pl.BlockSpec(memory_space=pltpu.SMEM),
        ],
        out_shape=[
            jax.ShapeDtypeStruct((n, 1), jnp.int32),
            jax.ShapeDtypeStruct((1, 1), jnp.float32),
            jax.ShapeDtypeStruct((1, 1), jnp.float32),
        ],
        scratch_shapes=[
            pltpu.SMEM((1, 1), jnp.float32),
            pltpu.VMEM((1, kp), jnp.float32),
        ],
    )(xsq, x, wt_pad, wsq_pad, codes, iotaf)
    return idx2.reshape(n), loss[0, 0], perp[0, 0]


def _sc_gather(table, idx):
    b = idx.shape[0]
    d = table.shape[1]
    b_per_w = b // _SC_WORKERS
    n_chunks = b_per_w // _SC_CHUNK
    mesh = plsc.VectorSubcoreMesh(core_axis_name="c", subcore_axis_name="s")

    @functools.partial(
        pl.kernel,
        out_type=jax.ShapeDtypeStruct((b, d), jnp.float32),
        mesh=mesh,
        scratch_types=[
            pltpu.VMEM((_SC_CHUNK,), jnp.int32),
            pltpu.VMEM((_SC_CHUNK, d), jnp.float32),
            pltpu.SemaphoreType.DMA,
        ],
    )
    def gather_kernel(idx_hbm, table_hbm, out_hbm, idx_v, rows_v, sem):
        wid = lax.axis_index("s") * _SC_CORES + lax.axis_index("c")
        base = wid * b_per_w
        for c in range(n_chunks):
            off = base + c * _SC_CHUNK
            pltpu.sync_copy(idx_hbm.at[pl.ds(off, _SC_CHUNK)], idx_v)
            pltpu.async_copy(table_hbm.at[idx_v], rows_v, sem).wait()
            pltpu.sync_copy(rows_v, out_hbm.at[pl.ds(off, _SC_CHUNK)])

    return gather_kernel(idx, table)


def kernel(inputs, W):
    k, d = W.shape
    x = jnp.transpose(inputs, (0, 2, 3, 1))
    x_flat = x.reshape(-1, d)
    xsq = jnp.sum(x_flat ** 2, axis=1, keepdims=True)
    wsq = jnp.sum(W ** 2, axis=1)
    wt = 2.0 * W.T

    pads = (_PADW - _WIN, _PADW - _WIN, _PADW - (k - 2 * _WIN))
    wt_win, wsq_win, code_win = [], [], []
    for w in range(3):
        lo = w * _WIN
        hi = min((w + 1) * _WIN, k)
        wt_win.append(jnp.pad(wt[:, lo:hi], ((0, 0), (0, pads[w]))))
        wsq_win.append(jnp.pad(wsq[lo:hi], (0, pads[w]),
                               constant_values=jnp.inf))
        code_win.append(jnp.pad(jnp.arange(lo, hi, dtype=jnp.int32),
                                (0, pads[w]), constant_values=-1))
    wt_pad = jnp.concatenate(wt_win, axis=1)
    wsq_pad = jnp.concatenate(wsq_win).reshape(1, -1)
    codes = jnp.concatenate(code_win).reshape(1, -1)

    iotaf = jnp.arange(_PADW, dtype=jnp.float32).reshape(1, -1)
    idx, loss, perp = _tc_vq(x_flat, xsq, wt_pad, wsq_pad, codes, iotaf, k)
    quantized = _sc_gather(W, idx)
    q_nchw = jnp.transpose(quantized.reshape(x.shape), (0, 3, 1, 2))
    return loss, q_nchw, perp, idx
